# Initial kernel scaffold; baseline (speedup 1.0000x reference)
import jax
import jax.numpy as jnp
from jax import lax
from jax.experimental import pallas as pl
from jax.experimental.pallas import tpu as pltpu
from jax.experimental.pallas import tpu_sc as plsc

N_NODES = 10000
N_EDGES = 320000
FEAT = 128

NC = 2
NS = 16
NW = NC * NS

CHUNK = 128
CPW = -(-N_EDGES // (CHUNK * NW))
E_PAD = CPW * CHUNK * NW

ROWS_PER_TILE = -(-N_NODES // NS)
AGG_ROWS = ((N_NODES + NS) + NS - 1) // NS * NS
ZERO_ROWS = AGG_ROWS // NS


def _mm_hs_body(h_ref, wt_ref, b_ref, o_ref):
    o_ref[...] = (
        jnp.dot(h_ref[...], wt_ref[...], preferred_element_type=jnp.float32)
        + b_ref[...]
    )


def _mm_out_body(p_ref, w_ref, b_ref, o_ref):
    agg = p_ref[0] + p_ref[1]
    o_ref[...] = (
        jnp.dot(agg, w_ref[...], preferred_element_type=jnp.float32)
        + b_ref[...]
    )


def _sc_body(hs_hbm, src_hbm, dst_hbm, zeros_hbm, out_hbm,
             agg, src_v, dst_v, msg_v, sem):
    c = lax.axis_index("core")
    s = lax.axis_index("subcore")
    wid = s * NC + c

    pltpu.sync_copy(zeros_hbm, agg.at[pl.ds(s * ZERO_ROWS, ZERO_ROWS)])

    pltpu.sync_copy(src_hbm.at[pl.ds(wid * CPW, CPW)], src_v)
    pltpu.sync_copy(dst_hbm.at[pl.ds(wid * CPW, CPW)], dst_v)

    plsc.subcore_barrier()

    @pl.loop(0, CPW)
    def _(j):
        pltpu.async_copy(hs_hbm.at[src_v.at[j]], msg_v, sem).wait()
        pltpu.sync_copy(msg_v, agg.at[dst_v.at[j]], add=True)

    plsc.subcore_barrier()

    pltpu.sync_copy(
        agg.at[pl.ds(s * ROWS_PER_TILE, ROWS_PER_TILE)],
        out_hbm.at[c, pl.ds(s * ROWS_PER_TILE, ROWS_PER_TILE)],
    )


def kernel(h, adj, lin_W, lin_b, weight, bias):
    h = h.astype(jnp.float32)
    src = adj[0].astype(jnp.int32)
    dst = adj[1].astype(jnp.int32)

    pad = E_PAD - N_EDGES
    src_p = jnp.concatenate([src, jnp.zeros((pad,), jnp.int32)])
    dst_p = jnp.concatenate([dst, jnp.full((pad,), N_NODES, jnp.int32)])
    src_p = src_p.reshape(NW * CPW, CHUNK)
    dst_p = dst_p.reshape(NW * CPW, CHUNK)

    blk = 1000
    hs = pl.pallas_call(
        _mm_hs_body,
        grid=(N_NODES // blk,),
        in_specs=[
            pl.BlockSpec((blk, FEAT), lambda i: (i, 0)),
            pl.BlockSpec((FEAT, FEAT), lambda i: (0, 0)),
            pl.BlockSpec((1, FEAT), lambda i: (0, 0)),
        ],
        out_specs=pl.BlockSpec((blk, FEAT), lambda i: (i, 0)),
        out_shape=jax.ShapeDtypeStruct((N_NODES, FEAT), jnp.float32),
    )(h, lin_W.T, lin_b.reshape(1, FEAT))

    zeros = jnp.zeros((ZERO_ROWS, FEAT), jnp.float32)
    mesh = plsc.VectorSubcoreMesh(
        core_axis_name="core", subcore_axis_name="subcore")
    sc_call = pl.kernel(
        _sc_body,
        out_type=jax.ShapeDtypeStruct((NC, N_NODES, FEAT), jnp.float32),
        mesh=mesh,
        scratch_types=[
            pltpu.VMEM_SHARED((AGG_ROWS, FEAT), jnp.float32),
            pltpu.VMEM((CPW, CHUNK), jnp.int32),
            pltpu.VMEM((CPW, CHUNK), jnp.int32),
            pltpu.VMEM((CHUNK, FEAT), jnp.float32),
            pltpu.SemaphoreType.DMA,
        ],
    )
    partials = sc_call(hs, src_p, dst_p, zeros)

    out = pl.pallas_call(
        _mm_out_body,
        grid=(N_NODES // blk,),
        in_specs=[
            pl.BlockSpec((NC, blk, FEAT), lambda i: (0, i, 0)),
            pl.BlockSpec((FEAT, FEAT), lambda i: (0, 0)),
            pl.BlockSpec((1, FEAT), lambda i: (0, 0)),
        ],
        out_specs=pl.BlockSpec((blk, FEAT), lambda i: (i, 0)),
        out_shape=jax.ShapeDtypeStruct((N_NODES, FEAT), jnp.float32),
    )(partials, weight, bias.reshape(1, FEAT))
    return out


# speedup vs baseline: 3.4522x; 3.4522x over previous
import jax
import jax.numpy as jnp
from jax import lax
from jax.experimental import pallas as pl
from jax.experimental.pallas import tpu as pltpu
from jax.experimental.pallas import tpu_sc as plsc

N_NODES = 10000
N_EDGES = 320000
FEAT = 128

NC = 2
NS = 16
NW = NC * NS

CHUNK = 128
CPW = (-(-N_EDGES // (CHUNK * NW)) + 7) // 8 * 8
E_PAD = CPW * CHUNK * NW

ROWS_PER_TILE = (-(-N_NODES // NS) + 7) // 8 * 8
PART_ROWS = ROWS_PER_TILE * NS
AGG_ROWS = 10240
ZERO_ROWS = AGG_ROWS // NS


def _mm_hs_body(h_ref, wt_ref, b_ref, o_ref):
    o_ref[...] = (
        jnp.dot(h_ref[...], wt_ref[...], preferred_element_type=jnp.float32)
        + b_ref[...]
    )


def _mm_out_body(p_ref, w_ref, b_ref, o_ref):
    agg = p_ref[0] + p_ref[1]
    o_ref[...] = (
        jnp.dot(agg, w_ref[...], preferred_element_type=jnp.float32)
        + b_ref[...]
    )


def _sc_body(hs_hbm, src_hbm, dst_hbm, zeros_hbm, out_hbm,
             agg, src_v, dst_v, msg_v, sem):
    c = lax.axis_index("core")
    s = lax.axis_index("subcore")
    wid = s * NC + c

    pltpu.sync_copy(zeros_hbm, agg.at[pl.ds(s * ZERO_ROWS, ZERO_ROWS)])

    pltpu.sync_copy(src_hbm.at[pl.ds(wid * CPW, CPW)], src_v)
    pltpu.sync_copy(dst_hbm.at[pl.ds(wid * CPW, CPW)], dst_v)

    plsc.subcore_barrier()

    @pl.loop(0, CPW)
    def _(j):
        pltpu.async_copy(hs_hbm.at[src_v.at[j]], msg_v, sem).wait()
        pltpu.sync_copy(msg_v, agg.at[dst_v.at[j]], add=True)

    plsc.subcore_barrier()

    pltpu.sync_copy(
        agg.at[pl.ds(s * ROWS_PER_TILE, ROWS_PER_TILE)],
        out_hbm.at[c, pl.ds(s * ROWS_PER_TILE, ROWS_PER_TILE)],
    )


def kernel(h, adj, lin_W, lin_b, weight, bias):
    h = h.astype(jnp.float32)
    src = adj[0].astype(jnp.int32)
    dst = adj[1].astype(jnp.int32)

    pad = E_PAD - N_EDGES
    src_p = jnp.concatenate([src, jnp.zeros((pad,), jnp.int32)])
    dst_p = jnp.concatenate([dst, jnp.full((pad,), N_NODES, jnp.int32)])
    src_p = src_p.reshape(NW * CPW, CHUNK)
    dst_p = dst_p.reshape(NW * CPW, CHUNK)

    blk = 1000
    hs = pl.pallas_call(
        _mm_hs_body,
        grid=(N_NODES // blk,),
        in_specs=[
            pl.BlockSpec((blk, FEAT), lambda i: (i, 0)),
            pl.BlockSpec((FEAT, FEAT), lambda i: (0, 0)),
            pl.BlockSpec((1, FEAT), lambda i: (0, 0)),
        ],
        out_specs=pl.BlockSpec((blk, FEAT), lambda i: (i, 0)),
        out_shape=jax.ShapeDtypeStruct((N_NODES, FEAT), jnp.float32),
    )(h, lin_W.T, lin_b.reshape(1, FEAT))

    zeros = jnp.zeros((ZERO_ROWS, FEAT), jnp.float32)
    mesh = plsc.VectorSubcoreMesh(
        core_axis_name="core", subcore_axis_name="subcore")
    sc_call = pl.kernel(
        _sc_body,
        out_type=jax.ShapeDtypeStruct((NC, PART_ROWS, FEAT), jnp.float32),
        mesh=mesh,
        scratch_types=[
            pltpu.VMEM_SHARED((AGG_ROWS, FEAT), jnp.float32),
            pltpu.VMEM((CPW, CHUNK), jnp.int32),
            pltpu.VMEM((CPW, CHUNK), jnp.int32),
            pltpu.VMEM((CHUNK, FEAT), jnp.float32),
            pltpu.SemaphoreType.DMA,
        ],
    )
    partials = sc_call(hs, src_p, dst_p, zeros)

    out = pl.pallas_call(
        _mm_out_body,
        grid=(N_NODES // blk,),
        in_specs=[
            pl.BlockSpec((NC, blk, FEAT), lambda i: (0, i, 0)),
            pl.BlockSpec((FEAT, FEAT), lambda i: (0, 0)),
            pl.BlockSpec((1, FEAT), lambda i: (0, 0)),
        ],
        out_specs=pl.BlockSpec((blk, FEAT), lambda i: (i, 0)),
        out_shape=jax.ShapeDtypeStruct((N_NODES, FEAT), jnp.float32),
    )(partials, weight, bias.reshape(1, FEAT))
    return out


# speedup vs baseline: 3.8733x; 1.1220x over previous
import jax
import jax.numpy as jnp
from jax import lax
from jax.experimental import pallas as pl
from jax.experimental.pallas import tpu as pltpu
from jax.experimental.pallas import tpu_sc as plsc

N_NODES = 10000
N_EDGES = 320000
FEAT = 128

NC = 2
NS = 16
NW = NC * NS

CHUNK = 128
CPW = (-(-N_EDGES // (CHUNK * NW)) + 7) // 8 * 8
E_PAD = CPW * CHUNK * NW

ROWS_PER_TILE = (-(-N_NODES // NS) + 7) // 8 * 8
PART_ROWS = ROWS_PER_TILE * NS
AGG_ROWS = PART_ROWS
ZERO_ROWS = AGG_ROWS // NS
HALF = CPW // 2


def _mm_hs_body(h_ref, wt_ref, b_ref, o_ref):
    o_ref[...] = (
        jnp.dot(h_ref[...], wt_ref[...], preferred_element_type=jnp.float32)
        + b_ref[...]
    )


def _mm_out_body(p_ref, w_ref, b_ref, o_ref):
    agg = p_ref[0] + p_ref[1]
    o_ref[...] = (
        jnp.dot(agg, w_ref[...], preferred_element_type=jnp.float32)
        + b_ref[...]
    )


def _sc_body(hs_hbm, src_hbm, dst_hbm, zeros_hbm, out_hbm,
             agg, src_v, dst_v, msg0, msg1, sem0, sem1):
    c = lax.axis_index("core")
    s = lax.axis_index("subcore")
    wid = s * NC + c

    pltpu.sync_copy(zeros_hbm, agg.at[pl.ds(s * ZERO_ROWS, ZERO_ROWS)])

    plsc.subcore_barrier()

    for h in range(CPW // HALF):
        pltpu.sync_copy(
            src_hbm.at[pl.ds(wid * CPW + h * HALF, HALF)], src_v)
        pltpu.sync_copy(
            dst_hbm.at[pl.ds(wid * CPW + h * HALF, HALF)], dst_v)

        pltpu.async_copy(hs_hbm.at[src_v.at[0]], msg0, sem0)
        pltpu.async_copy(hs_hbm.at[src_v.at[1]], msg1, sem1)

        @pl.loop(0, HALF, step=2)
        def _(g):
            for buf, sem, off in ((msg0, sem0, 0), (msg1, sem1, 1)):
                j = g + off
                pltpu.make_async_copy(
                    hs_hbm.at[pl.ds(0, CHUNK)], buf, sem).wait()
                pltpu.sync_copy(buf, agg.at[dst_v.at[j]], add=True)

                @pl.when(j + 2 < HALF)
                def _():
                    pltpu.async_copy(hs_hbm.at[src_v.at[j + 2]], buf, sem)

    plsc.subcore_barrier()

    pltpu.sync_copy(
        agg.at[pl.ds(s * ROWS_PER_TILE, ROWS_PER_TILE)],
        out_hbm.at[c, pl.ds(s * ROWS_PER_TILE, ROWS_PER_TILE)],
    )


def kernel(h, adj, lin_W, lin_b, weight, bias):
    h = h.astype(jnp.float32)
    src = adj[0].astype(jnp.int32)
    dst = adj[1].astype(jnp.int32)

    pad = E_PAD - N_EDGES
    src_p = jnp.concatenate([src, jnp.zeros((pad,), jnp.int32)])
    dst_p = jnp.concatenate([dst, jnp.full((pad,), N_NODES, jnp.int32)])
    src_p = src_p.reshape(NW * CPW, CHUNK)
    dst_p = dst_p.reshape(NW * CPW, CHUNK)

    blk = 1000
    hs = pl.pallas_call(
        _mm_hs_body,
        grid=(N_NODES // blk,),
        in_specs=[
            pl.BlockSpec((blk, FEAT), lambda i: (i, 0)),
            pl.BlockSpec((FEAT, FEAT), lambda i: (0, 0)),
            pl.BlockSpec((1, FEAT), lambda i: (0, 0)),
        ],
        out_specs=pl.BlockSpec((blk, FEAT), lambda i: (i, 0)),
        out_shape=jax.ShapeDtypeStruct((N_NODES, FEAT), jnp.float32),
    )(h, lin_W.T, lin_b.reshape(1, FEAT))

    zeros = jnp.zeros((ZERO_ROWS, FEAT), jnp.float32)
    mesh = plsc.VectorSubcoreMesh(
        core_axis_name="core", subcore_axis_name="subcore")
    sc_call = pl.kernel(
        _sc_body,
        out_type=jax.ShapeDtypeStruct((NC, PART_ROWS, FEAT), jnp.float32),
        mesh=mesh,
        scratch_types=[
            pltpu.VMEM_SHARED((AGG_ROWS, FEAT), jnp.float32),
            pltpu.VMEM((HALF, CHUNK), jnp.int32),
            pltpu.VMEM((HALF, CHUNK), jnp.int32),
            pltpu.VMEM((CHUNK, FEAT), jnp.float32),
            pltpu.VMEM((CHUNK, FEAT), jnp.float32),
            pltpu.SemaphoreType.DMA,
            pltpu.SemaphoreType.DMA,
        ],
    )
    partials = sc_call(hs, src_p, dst_p, zeros)

    out = pl.pallas_call(
        _mm_out_body,
        grid=(N_NODES // blk,),
        in_specs=[
            pl.BlockSpec((NC, blk, FEAT), lambda i: (0, i, 0)),
            pl.BlockSpec((FEAT, FEAT), lambda i: (0, 0)),
            pl.BlockSpec((1, FEAT), lambda i: (0, 0)),
        ],
        out_specs=pl.BlockSpec((blk, FEAT), lambda i: (i, 0)),
        out_shape=jax.ShapeDtypeStruct((N_NODES, FEAT), jnp.float32),
    )(partials, weight, bias.reshape(1, FEAT))
    return out


# speedup vs baseline: 3.8799x; 1.0017x over previous
import jax
import jax.numpy as jnp
from jax import lax
from jax.experimental import pallas as pl
from jax.experimental.pallas import tpu as pltpu
from jax.experimental.pallas import tpu_sc as plsc

N_NODES = 10000
N_EDGES = 320000
FEAT = 128

NC = 2
NS = 16
NW = NC * NS

CHUNK = 64
NBUF = 4
CPW = (-(-N_EDGES // (CHUNK * NW)) + 7) // 8 * 8
E_PAD = CPW * CHUNK * NW

ROWS_PER_TILE = (-(-N_NODES // NS) + 7) // 8 * 8
PART_ROWS = ROWS_PER_TILE * NS
AGG_ROWS = PART_ROWS
ZERO_ROWS = AGG_ROWS // NS
HALF = CPW // 4


def _mm_hs_body(h_ref, wt_ref, b_ref, o_ref):
    o_ref[...] = (
        jnp.dot(h_ref[...], wt_ref[...], preferred_element_type=jnp.float32)
        + b_ref[...]
    )


def _mm_out_body(p_ref, w_ref, b_ref, o_ref):
    agg = p_ref[0] + p_ref[1]
    o_ref[...] = (
        jnp.dot(agg, w_ref[...], preferred_element_type=jnp.float32)
        + b_ref[...]
    )


def _sc_body(hs_hbm, src_hbm, dst_hbm, zeros_hbm, out_hbm,
             agg, src_v, dst_v, msgs, gsems, ssems):
    c = lax.axis_index("core")
    s = lax.axis_index("subcore")
    wid = s * NC + c

    pltpu.sync_copy(zeros_hbm, agg.at[pl.ds(s * ZERO_ROWS, ZERO_ROWS)])

    plsc.subcore_barrier()

    def wait_gather(b):
        pltpu.make_async_copy(
            hs_hbm.at[pl.ds(0, CHUNK)], msgs[b], gsems[b]).wait()

    def wait_scatter(b):
        pltpu.make_async_copy(
            msgs[b], agg.at[pl.ds(0, CHUNK)], ssems[b]).wait()

    for h in range(CPW // HALF):
        pltpu.sync_copy(
            src_hbm.at[pl.ds(wid * CPW + h * HALF, HALF)], src_v)
        pltpu.sync_copy(
            dst_hbm.at[pl.ds(wid * CPW + h * HALF, HALF)], dst_v)

        pltpu.async_copy(hs_hbm.at[src_v.at[0]], msgs[0], gsems[0])
        pltpu.async_copy(hs_hbm.at[src_v.at[1]], msgs[1], gsems[1])

        @pl.loop(0, HALF, step=NBUF)
        def _(g):
            for b in range(NBUF):
                j = g + b
                bn = (b + 2) % NBUF

                @pl.when(j >= 2)
                def _():
                    wait_scatter(bn)

                @pl.when(j + 2 < HALF)
                def _():
                    pltpu.async_copy(
                        hs_hbm.at[src_v.at[j + 2]], msgs[bn], gsems[bn])

                wait_gather(b)
                pltpu.async_copy(
                    msgs[b], agg.at[dst_v.at[j]], ssems[b], add=True)

        wait_scatter((HALF - 2) % NBUF)
        wait_scatter((HALF - 1) % NBUF)

    plsc.subcore_barrier()

    pltpu.sync_copy(
        agg.at[pl.ds(s * ROWS_PER_TILE, ROWS_PER_TILE)],
        out_hbm.at[c, pl.ds(s * ROWS_PER_TILE, ROWS_PER_TILE)],
    )


def kernel(h, adj, lin_W, lin_b, weight, bias):
    h = h.astype(jnp.float32)
    src = adj[0].astype(jnp.int32)
    dst = adj[1].astype(jnp.int32)

    pad = E_PAD - N_EDGES
    src_p = jnp.concatenate([src, jnp.zeros((pad,), jnp.int32)])
    dst_p = jnp.concatenate([dst, jnp.full((pad,), N_NODES, jnp.int32)])
    src_p = src_p.reshape(NW * CPW, CHUNK)
    dst_p = dst_p.reshape(NW * CPW, CHUNK)

    blk = 1000
    hs = pl.pallas_call(
        _mm_hs_body,
        grid=(N_NODES // blk,),
        in_specs=[
            pl.BlockSpec((blk, FEAT), lambda i: (i, 0)),
            pl.BlockSpec((FEAT, FEAT), lambda i: (0, 0)),
            pl.BlockSpec((1, FEAT), lambda i: (0, 0)),
        ],
        out_specs=pl.BlockSpec((blk, FEAT), lambda i: (i, 0)),
        out_shape=jax.ShapeDtypeStruct((N_NODES, FEAT), jnp.float32),
    )(h, lin_W.T, lin_b.reshape(1, FEAT))

    zeros = jnp.zeros((ZERO_ROWS, FEAT), jnp.float32)
    mesh = plsc.VectorSubcoreMesh(
        core_axis_name="core", subcore_axis_name="subcore")
    sc_call = pl.kernel(
        _sc_body,
        out_type=jax.ShapeDtypeStruct((NC, PART_ROWS, FEAT), jnp.float32),
        mesh=mesh,
        scratch_types=[
            pltpu.VMEM_SHARED((AGG_ROWS, FEAT), jnp.float32),
            pltpu.VMEM((HALF, CHUNK), jnp.int32),
            pltpu.VMEM((HALF, CHUNK), jnp.int32),
            [pltpu.VMEM((CHUNK, FEAT), jnp.float32) for _ in range(NBUF)],
            [pltpu.SemaphoreType.DMA for _ in range(NBUF)],
            [pltpu.SemaphoreType.DMA for _ in range(NBUF)],
        ],
    )
    partials = sc_call(hs, src_p, dst_p, zeros)

    out = pl.pallas_call(
        _mm_out_body,
        grid=(N_NODES // blk,),
        in_specs=[
            pl.BlockSpec((NC, blk, FEAT), lambda i: (0, i, 0)),
            pl.BlockSpec((FEAT, FEAT), lambda i: (0, 0)),
            pl.BlockSpec((1, FEAT), lambda i: (0, 0)),
        ],
        out_specs=pl.BlockSpec((blk, FEAT), lambda i: (i, 0)),
        out_shape=jax.ShapeDtypeStruct((N_NODES, FEAT), jnp.float32),
    )(partials, weight, bias.reshape(1, FEAT))
    return out
